# baseline (device time: 65905 ns/iter reference)
import jax
import jax.numpy as jnp
from jax import lax
from jax.experimental import pallas as pl
from jax.experimental.pallas import tpu as pltpu

N_DEV = 16
N_LAYERS = 3
N_BLOCKS = 4
CHUNKS_PER_BLOCK = N_DEV // N_BLOCKS


def kernel(x, Win0, Wout0, Win1, Wout1, Win2, Wout2):
    b, d = x.shape
    cb = b // N_DEV
    bb = b // N_BLOCKS

    def body(x_ref, win0_ref, wout0_ref, win1_ref, wout1_ref,
             win2_ref, wout2_ref, out_ref, partial_ref, rs_ref,
             xnext_ref, rs_send_sems, rs_recv_sems,
             ag_send_sems, ag_recv_sems):
        my_i = lax.axis_index("i")

        barrier_sem = pltpu.get_barrier_semaphore()
        for k in range(1, N_DEV):
            pl.semaphore_signal(
                barrier_sem, inc=1,
                device_id=((my_i + k) % N_DEV,),
                device_id_type=pl.DeviceIdType.MESH,
            )

        def my_rows():
            return pl.ds(my_i * cb, cb)

        def ag_wait_chunk(c):
            @pl.when(c != my_i)
            def _():
                pltpu.make_async_remote_copy(
                    src_ref=xnext_ref.at[pl.ds(c * cb, cb), :],
                    dst_ref=xnext_ref.at[pl.ds(c * cb, cb), :],
                    send_sem=ag_send_sems.at[0],
                    recv_sem=ag_recv_sems.at[c],
                    device_id=(c,),
                    device_id_type=pl.DeviceIdType.MESH,
                ).wait_recv()

        for layer, (win_ref, wout_ref) in enumerate(((win0_ref, wout0_ref),
                                                     (win1_ref, wout1_ref),
                                                     (win2_ref, wout2_ref))):
            win = win_ref[...]
            wout = wout_ref[...]
            for g in range(N_BLOCKS):
                rows = pl.ds(g * bb, bb)
                chunks = range(g * CHUNKS_PER_BLOCK,
                               (g + 1) * CHUNKS_PER_BLOCK)

                if layer > 0:
                    for c in chunks:
                        ag_wait_chunk(c)
                    x_g = xnext_ref[rows, :]
                else:
                    x_g = x_ref[rows, :]

                h_g = jnp.maximum(
                    jnp.dot(x_g, win, preferred_element_type=jnp.float32),
                    0.0,
                )
                partial_ref[rows, :] = jnp.dot(
                    h_g, wout, preferred_element_type=jnp.float32
                ).astype(jnp.bfloat16)

                if layer == 0 and g == 0:
                    pl.semaphore_wait(barrier_sem, N_DEV - 1)

                for c in chunks:
                    @pl.when(c != my_i)
                    def _(c=c):
                        pltpu.make_async_remote_copy(
                            src_ref=partial_ref.at[pl.ds(c * cb, cb), :],
                            dst_ref=rs_ref.at[my_i],
                            send_sem=rs_send_sems.at[c],
                            recv_sem=rs_recv_sems.at[my_i],
                            device_id=(c,),
                            device_id_type=pl.DeviceIdType.MESH,
                        ).start()

                @pl.when(my_i // CHUNKS_PER_BLOCK == g)
                def _():
                    reduced = partial_ref[my_rows(), :].astype(jnp.float32)
                    for k in range(1, N_DEV):
                        src = (my_i - k) % N_DEV
                        pltpu.make_async_remote_copy(
                            src_ref=partial_ref.at[my_rows(), :],
                            dst_ref=rs_ref.at[src],
                            send_sem=rs_send_sems.at[0],
                            recv_sem=rs_recv_sems.at[src],
                            device_id=(src,),
                            device_id_type=pl.DeviceIdType.MESH,
                        ).wait_recv()
                        reduced = reduced + rs_ref[src].astype(jnp.float32)
                    xnext_ref[my_rows(), :] = reduced.astype(jnp.bfloat16)
                    for k in range(1, N_DEV):
                        dest = (my_i + k) % N_DEV
                        pltpu.make_async_remote_copy(
                            src_ref=xnext_ref.at[my_rows(), :],
                            dst_ref=xnext_ref.at[my_rows(), :],
                            send_sem=ag_send_sems.at[k - 1],
                            recv_sem=ag_recv_sems.at[my_i],
                            device_id=(dest,),
                            device_id_type=pl.DeviceIdType.MESH,
                        ).start()

            for c in range(N_DEV):
                @pl.when(c != my_i)
                def _(c=c):
                    pltpu.make_async_remote_copy(
                        src_ref=partial_ref.at[pl.ds(c * cb, cb), :],
                        dst_ref=rs_ref.at[my_i],
                        send_sem=rs_send_sems.at[c],
                        recv_sem=rs_recv_sems.at[my_i],
                        device_id=(c,),
                        device_id_type=pl.DeviceIdType.MESH,
                    ).wait_send()
            for k in range(1, N_DEV):
                pltpu.make_async_remote_copy(
                    src_ref=xnext_ref.at[my_rows(), :],
                    dst_ref=xnext_ref.at[my_rows(), :],
                    send_sem=ag_send_sems.at[k - 1],
                    recv_sem=ag_recv_sems.at[my_i],
                    device_id=((my_i + k) % N_DEV,),
                    device_id_type=pl.DeviceIdType.MESH,
                ).wait_send()

        for g in range(N_BLOCKS):
            for c in range(g * CHUNKS_PER_BLOCK, (g + 1) * CHUNKS_PER_BLOCK):
                ag_wait_chunk(c)
            rows = pl.ds(g * bb, bb)
            out_ref[rows, :] = xnext_ref[rows, :].astype(jnp.float32)

    return pl.pallas_call(
        body,
        out_shape=jax.ShapeDtypeStruct((b, d), jnp.float32),
        in_specs=[pl.BlockSpec(memory_space=pltpu.VMEM)] * 7,
        out_specs=pl.BlockSpec(memory_space=pltpu.VMEM),
        scratch_shapes=[
            pltpu.VMEM((b, d), jnp.bfloat16),
            pltpu.VMEM((N_DEV, cb, d), jnp.bfloat16),
            pltpu.VMEM((b, d), jnp.bfloat16),
            pltpu.SemaphoreType.DMA((N_DEV,)),
            pltpu.SemaphoreType.DMA((N_DEV,)),
            pltpu.SemaphoreType.DMA((N_DEV - 1,)),
            pltpu.SemaphoreType.DMA((N_DEV,)),
        ],
        compiler_params=pltpu.CompilerParams(collective_id=0),
    )(x, Win0, Wout0, Win1, Wout1, Win2, Wout2)


# device time: 41275 ns/iter; 1.5967x vs baseline; 1.5967x over previous
import jax
import jax.numpy as jnp
from jax import lax
from jax.experimental import pallas as pl
from jax.experimental.pallas import tpu as pltpu

N_DEV = 16
N_LAYERS = 3
N_WAVES = 2


def kernel(x, Win0, Wout0, Win1, Wout1, Win2, Wout2):
    b, d = x.shape
    cb = b // N_DEV
    wc = d // N_WAVES

    def body(x_ref, win0_ref, wout0_ref, win1_ref, wout1_ref,
             win2_ref, wout2_ref, out_ref, partial_ref, rs_ref,
             xnext_ref, send_sems, recv_sems):
        my_i = lax.axis_index("i")

        barrier_sem = pltpu.get_barrier_semaphore()
        for k in range(1, N_DEV):
            pl.semaphore_signal(
                barrier_sem, inc=1,
                device_id=((my_i + k) % N_DEV,),
                device_id_type=pl.DeviceIdType.MESH,
            )

        def half(i, w):
            return (pl.ds(i * cb, cb), pl.ds(w * wc, wc))

        def rs_send(k, w):
            dest = (my_i + k) % N_DEV
            rdma = pltpu.make_async_remote_copy(
                src_ref=partial_ref.at[half(dest, w)],
                dst_ref=rs_ref.at[w, my_i],
                send_sem=send_sems.at[0, w, k - 1],
                recv_sem=recv_sems.at[0, w, k - 1],
                device_id=(dest,),
                device_id_type=pl.DeviceIdType.MESH,
            )
            rdma.start()
            return rdma

        def rs_wait_reduce(w):
            reduced = partial_ref[half(my_i, w)].astype(jnp.float32)
            for k in range(1, N_DEV):
                src = (my_i - k) % N_DEV
                pltpu.make_async_remote_copy(
                    src_ref=partial_ref.at[half(my_i, w)],
                    dst_ref=rs_ref.at[w, src],
                    send_sem=send_sems.at[0, w, k - 1],
                    recv_sem=recv_sems.at[0, w, k - 1],
                    device_id=(src,),
                    device_id_type=pl.DeviceIdType.MESH,
                ).wait_recv()
                reduced = reduced + rs_ref[w, src].astype(jnp.float32)
            return reduced

        def ag_send(k, w):
            dest = (my_i + k) % N_DEV
            rdma = pltpu.make_async_remote_copy(
                src_ref=xnext_ref.at[half(my_i, w)],
                dst_ref=xnext_ref.at[half(my_i, w)],
                send_sem=send_sems.at[1, w, k - 1],
                recv_sem=recv_sems.at[1, w, k - 1],
                device_id=(dest,),
                device_id_type=pl.DeviceIdType.MESH,
            )
            rdma.start()
            return rdma

        def ag_wait(w):
            for k in range(1, N_DEV):
                src = (my_i - k) % N_DEV
                pltpu.make_async_remote_copy(
                    src_ref=xnext_ref.at[half(src, w)],
                    dst_ref=xnext_ref.at[half(src, w)],
                    send_sem=send_sems.at[1, w, k - 1],
                    recv_sem=recv_sems.at[1, w, k - 1],
                    device_id=(src,),
                    device_id_type=pl.DeviceIdType.MESH,
                ).wait_recv()

        x_cur = x_ref[...]
        for layer, (win_ref, wout_ref) in enumerate(((win0_ref, wout0_ref),
                                                     (win1_ref, wout1_ref),
                                                     (win2_ref, wout2_ref))):
            h = jnp.maximum(
                jnp.dot(x_cur, win_ref[...], preferred_element_type=jnp.float32),
                0.0,
            )
            partial_ref[...] = jnp.dot(
                h, wout_ref[...], preferred_element_type=jnp.float32
            ).astype(jnp.bfloat16)
            if layer == 0:
                pl.semaphore_wait(barrier_sem, N_DEV - 1)

            sends = []
            for k in range(1, N_DEV):
                sends.append(rs_send(k, 0))
            for k in range(1, N_DEV):
                sends.append(rs_send(k, 1))

            xnext_ref[half(my_i, 0)] = rs_wait_reduce(0).astype(jnp.bfloat16)
            for k in range(1, N_DEV):
                sends.append(ag_send(k, 0))

            xnext_ref[half(my_i, 1)] = rs_wait_reduce(1).astype(jnp.bfloat16)
            for k in range(1, N_DEV):
                sends.append(ag_send(k, 1))

            ag_wait(0)
            ag_wait(1)

            for rdma in sends:
                rdma.wait_send()

            x_cur = xnext_ref[...]
        out_ref[...] = x_cur.astype(jnp.float32)

    return pl.pallas_call(
        body,
        out_shape=jax.ShapeDtypeStruct((b, d), jnp.float32),
        in_specs=[pl.BlockSpec(memory_space=pltpu.VMEM)] * 7,
        out_specs=pl.BlockSpec(memory_space=pltpu.VMEM),
        scratch_shapes=[
            pltpu.VMEM((b, d), jnp.bfloat16),
            pltpu.VMEM((N_WAVES, N_DEV, cb, wc), jnp.bfloat16),
            pltpu.VMEM((b, d), jnp.bfloat16),
            pltpu.SemaphoreType.DMA((2, N_WAVES, N_DEV - 1)),
            pltpu.SemaphoreType.DMA((2, N_WAVES, N_DEV - 1)),
        ],
        compiler_params=pltpu.CompilerParams(collective_id=0),
    )(x, Win0, Wout0, Win1, Wout1, Win2, Wout2)


# device time: 37509 ns/iter; 1.7570x vs baseline; 1.1004x over previous
import jax
import jax.numpy as jnp
from jax import lax
from jax.experimental import pallas as pl
from jax.experimental.pallas import tpu as pltpu

N_DEV = 16
N_LAYERS = 3


def kernel(x, Win0, Wout0, Win1, Wout1, Win2, Wout2):
    b, d = x.shape
    cb = b // N_DEV

    def body(x_ref, win0_ref, wout0_ref, win1_ref, wout1_ref,
             win2_ref, wout2_ref, out_ref, partial_ref, rs_ref,
             xnext_ref, send_sems, recv_sems):
        my_i = lax.axis_index("i")

        barrier_sem = pltpu.get_barrier_semaphore()
        for k in range(1, N_DEV):
            pl.semaphore_signal(
                barrier_sem, inc=1,
                device_id=((my_i + k) % N_DEV,),
                device_id_type=pl.DeviceIdType.MESH,
            )

        def chunk_rows(i):
            return pl.ds(i * cb, cb)

        x_cur = x_ref[...]
        for layer, (win_ref, wout_ref) in enumerate(((win0_ref, wout0_ref),
                                                     (win1_ref, wout1_ref),
                                                     (win2_ref, wout2_ref))):
            h = jnp.maximum(
                jnp.dot(x_cur, win_ref[...], preferred_element_type=jnp.float32),
                0.0,
            )
            partial_ref[...] = jnp.dot(
                h, wout_ref[...], preferred_element_type=jnp.float32
            ).astype(jnp.bfloat16)
            if layer == 0:
                pl.semaphore_wait(barrier_sem, N_DEV - 1)

            rs_sends = []
            for k in range(1, N_DEV):
                dest = (my_i + k) % N_DEV
                rdma = pltpu.make_async_remote_copy(
                    src_ref=partial_ref.at[chunk_rows(dest), :],
                    dst_ref=rs_ref.at[my_i],
                    send_sem=send_sems.at[0, k - 1],
                    recv_sem=recv_sems.at[0, k - 1],
                    device_id=(dest,),
                    device_id_type=pl.DeviceIdType.MESH,
                )
                rdma.start()
                rs_sends.append(rdma)

            reduced = partial_ref[chunk_rows(my_i), :].astype(jnp.float32)
            for k in range(1, N_DEV):
                src = (my_i - k) % N_DEV
                pltpu.make_async_remote_copy(
                    src_ref=partial_ref.at[chunk_rows(my_i), :],
                    dst_ref=rs_ref.at[src],
                    send_sem=send_sems.at[0, k - 1],
                    recv_sem=recv_sems.at[0, k - 1],
                    device_id=(src,),
                    device_id_type=pl.DeviceIdType.MESH,
                ).wait_recv()
                reduced = reduced + rs_ref[src].astype(jnp.float32)
            xnext_ref[chunk_rows(my_i), :] = reduced.astype(jnp.bfloat16)

            ag_sends = []
            for k in range(1, N_DEV):
                dest = (my_i + k) % N_DEV
                rdma = pltpu.make_async_remote_copy(
                    src_ref=xnext_ref.at[chunk_rows(my_i), :],
                    dst_ref=xnext_ref.at[chunk_rows(my_i), :],
                    send_sem=send_sems.at[1, k - 1],
                    recv_sem=recv_sems.at[1, k - 1],
                    device_id=(dest,),
                    device_id_type=pl.DeviceIdType.MESH,
                )
                rdma.start()
                ag_sends.append(rdma)

            for k in range(1, N_DEV):
                src = (my_i - k) % N_DEV
                pltpu.make_async_remote_copy(
                    src_ref=xnext_ref.at[chunk_rows(src), :],
                    dst_ref=xnext_ref.at[chunk_rows(src), :],
                    send_sem=send_sems.at[1, k - 1],
                    recv_sem=recv_sems.at[1, k - 1],
                    device_id=(src,),
                    device_id_type=pl.DeviceIdType.MESH,
                ).wait_recv()

            for rdma in rs_sends:
                rdma.wait_send()
            for rdma in ag_sends:
                rdma.wait_send()

            x_cur = xnext_ref[...]
        out_ref[...] = x_cur.astype(jnp.float32)

    return pl.pallas_call(
        body,
        out_shape=jax.ShapeDtypeStruct((b, d), jnp.float32),
        in_specs=[pl.BlockSpec(memory_space=pltpu.VMEM)] * 7,
        out_specs=pl.BlockSpec(memory_space=pltpu.VMEM),
        scratch_shapes=[
            pltpu.VMEM((b, d), jnp.bfloat16),
            pltpu.VMEM((N_DEV, cb, d), jnp.bfloat16),
            pltpu.VMEM((b, d), jnp.bfloat16),
            pltpu.SemaphoreType.DMA((2, N_DEV - 1)),
            pltpu.SemaphoreType.DMA((2, N_DEV - 1)),
        ],
        compiler_params=pltpu.CompilerParams(collective_id=0),
    )(x, Win0, Wout0, Win1, Wout1, Win2, Wout2)
